# baseline (speedup 1.0000x reference)
import jax
import jax.numpy as jnp
from jax import lax
from jax.experimental import pallas as pl
from jax.experimental.pallas import tpu as pltpu
from jax.experimental.pallas import tpu_sc as plsc

_EPS = 1e-12
_NUM = 2

_NC = 2
_NS = 16
_NW = _NC * _NS
_L = 16

_B = 4
_C = 96
_HW = 384 * 384
_TPW = _HW // _NW
_CH = 768
_NK = _TPW // _CH
_GRP = 128
_NG = _CH // _GRP
_GB = _GRP // 32


def _sc_body(qbf, cbq, pcnt, labels,
             buf0, buf1, cb_v, cnt, lbl_v, cb_s, sem0, sem1):
    cid = lax.axis_index("c")
    sid = lax.axis_index("s")
    wid = sid * _NC + cid
    base = wid * _TPW

    pltpu.sync_copy(cbq, cb_v)

    def _stage_w(g, carry):
        wv = cb_v[pl.ds(g * _L, _L)]
        for i in range(_L):
            cb_s[g * _L + i] = wv[i]
        return carry
    lax.fori_loop(0, 2 * _C // _L, _stage_w, 0)

    zf = jnp.zeros((_L,), jnp.float32)
    iota = lax.iota(jnp.int32, _L)
    for b in range(_B):
        cnt[b, :] = zf

    rings = (buf0, buf1)
    sems = (sem0, sem1)
    nchunks = _B * _NK

    def _dma(g):
        b, k = divmod(g, _NK)
        t0 = base + k * _CH
        return pltpu.make_async_copy(
            qbf.at[b, :, pl.ds(t0, _CH)], rings[g % 2], sems[g % 2])

    _dma(0).start()
    for b in range(_B):
        for k in range(_NK):
            g = b * _NK + k
            buf = rings[g % 2]
            _dma(g).wait()
            if g + 1 < nchunks:
                _dma(g + 1).start()

            def _grp(gi, carry, buf=buf, k=k, b=b):
                off = gi * _GRP

                def _dot(c, accs, off=off, buf=buf):
                    c0 = cb_s[c]
                    c1 = cb_s[_C + c]
                    s0s, s1s = accs
                    n0, n1 = [], []
                    for i in range(_GB):
                        qv = buf[c, pl.ds(off + 32 * i, 32)]
                        qe, qo = plsc.unpack(
                            qv, format=plsc.PackFormat.INTERLEAVED)
                        n0.append(s0s[2 * i] + qe * c0)
                        n0.append(s0s[2 * i + 1] + qo * c0)
                        n1.append(s1s[2 * i] + qe * c1)
                        n1.append(s1s[2 * i + 1] + qo * c1)
                    return (tuple(n0), tuple(n1))
                s0s, s1s = lax.fori_loop(
                    0, _C, _dot,
                    (tuple(zf for _ in range(2 * _GB)),
                     tuple(zf for _ in range(2 * _GB))))

                cs = zf
                pos = k * _CH + off
                for i in range(_GB):
                    for par in range(2):
                        d0 = 0.5 * (1.0 - s0s[2 * i + par])
                        d1 = 0.5 * (1.0 - s1s[2 * i + par])
                        m = d1 < d0
                        mf = jnp.where(m, 1.0, 0.0)
                        idx = iota * 2 + (pos + 32 * i + par)
                        plsc.store_scatter(lbl_v, [idx], jnp.where(m, 1, 0))
                        cs = cs + mf
                plsc.addupdate(cnt.at[b], cs)
                return carry
            lax.fori_loop(0, _NG, _grp, 0)

        pltpu.sync_copy(lbl_v, labels.at[b, pl.ds(base, _TPW)])

    pltpu.sync_copy(cnt, pcnt.at[wid])


_sc_pass_built = []


def _sc_pass_call(qbf, cbq):
    if not _sc_pass_built:
        _sc_pass_built.append(pl.kernel(
            _sc_body,
            out_type=[
                jax.ShapeDtypeStruct((_NW, _B, _L), jnp.float32),
                jax.ShapeDtypeStruct((_B, _HW), jnp.int32),
            ],
            mesh=plsc.VectorSubcoreMesh(
                core_axis_name="c", subcore_axis_name="s"),
            scratch_types=[
                pltpu.VMEM((_C, _CH), jnp.bfloat16),
                pltpu.VMEM((_C, _CH), jnp.bfloat16),
                pltpu.VMEM((2 * _C,), jnp.float32),
                pltpu.VMEM((_B, _L), jnp.float32),
                pltpu.VMEM((_TPW,), jnp.int32),
                pltpu.SMEM((2 * _C,), jnp.float32),
                pltpu.SemaphoreType.DMA,
                pltpu.SemaphoreType.DMA,
            ],
            compiler_params=pltpu.CompilerParams(
                use_tc_tiling_on_sc=False, needs_layout_passes=False),
        ))
    return _sc_pass_built[0](qbf, cbq)


def _normalize(x):
    n = jnp.linalg.norm(x, axis=1, keepdims=True)
    return x / jnp.maximum(n, _EPS)


def _cos_pair(A, B):
    return jnp.sum(_normalize(A) * _normalize(B), axis=1)


def _make_cbq(centers):
    ob = lax.optimization_barrier
    c2 = ob(centers * centers)
    ssum = ob(jnp.sum(c2, axis=1, keepdims=True))
    n = ob(jnp.sqrt(ssum))
    nm = ob(jnp.maximum(n, _EPS))
    c_hat = ob(centers / nm)
    cb16 = ob(c_hat.astype(jnp.bfloat16))
    return ob(cb16.astype(jnp.float32)).reshape(2 * _C)


def _make_qbf(Feat):
    ob = lax.optimization_barrier
    qs = []
    for b in range(_B):
        Fb = Feat[b]
        n = jnp.linalg.norm(Fb, axis=0, keepdims=True)
        nf = Fb / jnp.maximum(n, _EPS)
        qs.append(ob(nf.astype(jnp.bfloat16)))
    return jnp.stack(qs)


def _one_pass(Feat, qbf, centers):
    cbq = _make_cbq(centers)
    pcnt, labels = _sc_pass_call(qbf, cbq)
    n1 = jnp.sum(pcnt, axis=(0, 2))
    refs = jnp.arange(_NUM).reshape(_NUM, 1)
    centersIterout = 0.0
    CdistT = 0.0
    for b in range(_B):
        Fb = Feat[b].T
        lab_b = labels[b][None, :]
        mask_l = (lab_b == refs)[:, :, None].astype(jnp.float32)
        centersIter = jnp.sum(Fb[None, :, :] * mask_l, axis=1)
        counts = jnp.stack([_HW - n1[b], n1[b]])
        centersIter = centersIter / (counts[:, None] + 1.0)
        centersIterout = centersIterout + centersIter
        CdistT = CdistT + jnp.mean(_cos_pair(centersIter, centers))
    return centersIterout, CdistT, labels


def kernel(FeatureT, centerInit):
    Feat = FeatureT.reshape(_B, _C, _HW)
    qbf = _make_qbf(Feat)
    co, cd, labels = _one_pass(Feat, qbf, centerInit)
    count = jnp.asarray(1, jnp.int32)

    def cond_fun(carry):
        _, cd_c, _, cnt_c = carry
        return jnp.logical_not(
            jnp.logical_or(cd_c / _B < 0.01, (cnt_c + 1) > 3))

    def body_fun(carry):
        co_c, _, _, cnt_c = carry
        co2, cd2, lb2 = _one_pass(Feat, qbf, co_c / _B)
        return (co2, cd2, lb2, cnt_c + 1)

    co, cd, labels, count = lax.while_loop(
        cond_fun, body_fun, (co, cd, labels, count))

    centers = co / _B
    CurDist = cd / _B
    labels_onehot = jnp.eye(_NUM, dtype=jnp.float32)[labels.reshape(-1)]
    return (jax.lax.stop_gradient(centers), labels, labels_onehot, CurDist)


# speedup vs baseline: 1.8035x; 1.8035x over previous
import jax
import jax.numpy as jnp
from jax import lax
from jax.experimental import pallas as pl
from jax.experimental.pallas import tpu as pltpu
from jax.experimental.pallas import tpu_sc as plsc

_EPS = 1e-12
_NUM = 2

_NC = 2
_NS = 16
_NW = _NC * _NS
_L = 16

_B = 4
_C = 96
_HW = 384 * 384
_TPW = _HW // _NW
_CH = 768
_NK = _TPW // _CH
_GRP = 128
_NG = _CH // _GRP
_GB = _GRP // 32


def _sc_body(qbf, cbq, pcnt, labels,
             buf0, buf1, cb_v, cnt, lbl_v, cb_s, sem0, sem1):
    cid = lax.axis_index("c")
    sid = lax.axis_index("s")
    wid = sid * _NC + cid
    base = wid * _TPW

    pltpu.sync_copy(cbq, cb_v)

    def _stage_w(g, carry):
        wv = cb_v[pl.ds(g * _L, _L)]
        for i in range(_L):
            cb_s[g * _L + i] = wv[i]
        return carry
    lax.fori_loop(0, 2 * _C // _L, _stage_w, 0)

    zf = jnp.zeros((_L,), jnp.float32)
    iota = lax.iota(jnp.int32, _L)
    for b in range(_B):
        cnt[b, :] = zf

    rings = (buf0, buf1)
    sems = (sem0, sem1)
    nchunks = _B * _NK

    def _dma(g):
        b, k = divmod(g, _NK)
        t0 = base + k * _CH
        return pltpu.make_async_copy(
            qbf.at[b, :, pl.ds(t0, _CH)], rings[g % 2], sems[g % 2])

    _dma(0).start()
    for b in range(_B):
        for k in range(_NK):
            g = b * _NK + k
            buf = rings[g % 2]
            _dma(g).wait()
            if g + 1 < nchunks:
                _dma(g + 1).start()

            def _grp(gi, carry, buf=buf, k=k, b=b):
                off = gi * _GRP

                def _dot(c, accs, off=off, buf=buf):
                    c0 = cb_s[c]
                    c1 = cb_s[_C + c]
                    s0s, s1s = accs
                    n0, n1 = [], []
                    for i in range(_GB):
                        qv = buf[c, pl.ds(off + 32 * i, 32)]
                        qe, qo = plsc.unpack(
                            qv, format=plsc.PackFormat.INTERLEAVED)
                        n0.append(s0s[2 * i] + qe * c0)
                        n0.append(s0s[2 * i + 1] + qo * c0)
                        n1.append(s1s[2 * i] + qe * c1)
                        n1.append(s1s[2 * i + 1] + qo * c1)
                    return (tuple(n0), tuple(n1))
                s0s, s1s = lax.fori_loop(
                    0, _C, _dot,
                    (tuple(zf for _ in range(2 * _GB)),
                     tuple(zf for _ in range(2 * _GB))))

                cs = zf
                pos = k * _CH + off
                for i in range(_GB):
                    for par in range(2):
                        d0 = 0.5 * (1.0 - s0s[2 * i + par])
                        d1 = 0.5 * (1.0 - s1s[2 * i + par])
                        m = d1 < d0
                        mf = jnp.where(m, 1.0, 0.0)
                        idx = iota * 2 + (pos + 32 * i + par)
                        plsc.store_scatter(lbl_v, [idx], jnp.where(m, 1, 0))
                        cs = cs + mf
                plsc.addupdate(cnt.at[b], cs)
                return carry
            lax.fori_loop(0, _NG, _grp, 0)

        pltpu.sync_copy(lbl_v, labels.at[b, pl.ds(base, _TPW)])

    pltpu.sync_copy(cnt, pcnt.at[wid])


_sc_pass_built = []


def _sc_pass_call(qbf, cbq):
    if not _sc_pass_built:
        _sc_pass_built.append(pl.kernel(
            _sc_body,
            out_type=[
                jax.ShapeDtypeStruct((_NW, _B, _L), jnp.float32),
                jax.ShapeDtypeStruct((_B, _HW), jnp.int32),
            ],
            mesh=plsc.VectorSubcoreMesh(
                core_axis_name="c", subcore_axis_name="s"),
            scratch_types=[
                pltpu.VMEM((_C, _CH), jnp.bfloat16),
                pltpu.VMEM((_C, _CH), jnp.bfloat16),
                pltpu.VMEM((2 * _C,), jnp.float32),
                pltpu.VMEM((_B, _L), jnp.float32),
                pltpu.VMEM((_TPW,), jnp.int32),
                pltpu.SMEM((2 * _C,), jnp.float32),
                pltpu.SemaphoreType.DMA,
                pltpu.SemaphoreType.DMA,
            ],
            compiler_params=pltpu.CompilerParams(
                use_tc_tiling_on_sc=False, needs_layout_passes=False),
        ))
    return _sc_pass_built[0](qbf, cbq)


def _normalize(x):
    n = jnp.linalg.norm(x, axis=1, keepdims=True)
    return x / jnp.maximum(n, _EPS)


def _cos_pair(A, B):
    return jnp.sum(_normalize(A) * _normalize(B), axis=1)


def _make_cbq(centers):
    ob = lax.optimization_barrier
    c2 = ob(centers * centers)
    ssum = ob(jnp.sum(c2, axis=1, keepdims=True))
    n = ob(jnp.sqrt(ssum))
    nm = ob(jnp.maximum(n, _EPS))
    c_hat = ob(centers / nm)
    cb16 = ob(c_hat.astype(jnp.bfloat16))
    return ob(cb16.astype(jnp.float32)).reshape(2 * _C)


def _make_qbf(Feat):
    ob = lax.optimization_barrier
    qs = []
    for b in range(_B):
        Fb = Feat[b]
        n = jnp.linalg.norm(Fb, axis=0, keepdims=True)
        nf = Fb / jnp.maximum(n, _EPS)
        qs.append(ob(nf.astype(jnp.bfloat16)))
    return jnp.stack(qs)


def _one_pass(Feat, qbf, centers):
    cbq = _make_cbq(centers)
    pcnt, labels = _sc_pass_call(qbf, cbq)
    n1 = jnp.sum(pcnt, axis=(0, 2))
    refs = jnp.arange(_NUM).reshape(_NUM, 1)
    centersIterout = 0.0
    CdistT = 0.0
    for b in range(_B):
        Fb = Feat[b].T
        lab_b = labels[b][None, :]
        mask_l = (lab_b == refs)[:, :, None].astype(jnp.float32)
        centersIter = jnp.sum(Fb[None, :, :] * mask_l, axis=1)
        counts = jnp.stack([_HW - n1[b], n1[b]])
        centersIter = centersIter / (counts[:, None] + 1.0)
        centersIterout = centersIterout + centersIter
        CdistT = CdistT + jnp.mean(_cos_pair(centersIter, centers))
    return centersIterout, CdistT, labels


def kernel(FeatureT, centerInit):
    Feat = FeatureT.reshape(_B, _C, _HW)
    qbf = _make_qbf(Feat)
    co, cd, labels = _one_pass(Feat, qbf, centerInit)
    count = jnp.asarray(1, jnp.int32)

    def cond_fun(carry):
        _, cd_c, _, cnt_c = carry
        return jnp.logical_not(
            jnp.logical_or(cd_c / _B < 0.01, (cnt_c + 1) > 3))

    def body_fun(carry):
        co_c, _, _, cnt_c = carry
        co2, cd2, lb2 = _one_pass(Feat, qbf, co_c / _B)
        return (co2, cd2, lb2, cnt_c + 1)

    co, cd, labels, count = lax.while_loop(
        cond_fun, body_fun, (co, cd, labels, count))

    centers = co / _B
    CurDist = cd / _B
    lf = labels.reshape(-1).astype(jnp.float32)
    labels_onehot = jnp.stack([1.0 - lf, lf], axis=1)
    return (jax.lax.stop_gradient(centers), labels, labels_onehot, CurDist)
